# Initial kernel scaffold; baseline (speedup 1.0000x reference)
import jax
import jax.numpy as jnp
from jax.experimental import pallas as pl


def kernel(batch_idx, topk_idx, box_regression):
    raise NotImplementedError("write your pallas kernel here")


# speedup vs baseline: 1.1787x; 1.1787x over previous
import jax
import jax.numpy as jnp
from jax import lax
from jax.experimental import pallas as pl
from jax.experimental.pallas import tpu as pltpu
from jax.experimental.pallas import tpu_sc as plsc

K = 20000
KPAD = 20480
NW = 32
ROWS_PER_W = KPAD // NW
ELEMS_PER_W = ROWS_PER_W * 4


def _body(ti_hbm, br_hbm, out_hbm, ti_v, br_v, idx2_v, out_v):
    nc = 2
    wid = lax.axis_index("s") * nc + lax.axis_index("c")
    base = wid * ROWS_PER_W
    pltpu.sync_copy(ti_hbm, ti_v)
    pltpu.sync_copy(br_hbm, br_v)
    lanes = lax.iota(jnp.int32, 16)
    quarter = lanes >> 2
    comp = lanes & 3
    for j in range(ROWS_PER_W // 16):
        first = ti_v[pl.ds(base + j * 16, 16)]
        sec = plsc.load_gather(ti_v, [first])
        idx2_v[pl.ds(j * 16, 16)] = sec
    for j in range(ELEMS_PER_W // 16):
        rows = plsc.load_gather(idx2_v, [j * 4 + quarter])
        val = plsc.load_gather(br_v, [rows * 4 + comp])
        out_v[pl.ds(j * 16, 16)] = val
    pltpu.sync_copy(out_v, out_hbm.at[pl.ds(wid * ELEMS_PER_W, ELEMS_PER_W)])


@jax.jit
def _run(ti_pad, br_flat):
    mesh = plsc.VectorSubcoreMesh(
        core_axis_name="c", subcore_axis_name="s", num_cores=2, num_subcores=16
    )
    f = pl.kernel(
        _body,
        out_type=jax.ShapeDtypeStruct((KPAD * 4,), jnp.float32),
        mesh=mesh,
        scratch_types=[
            pltpu.VMEM((KPAD,), jnp.int32),
            pltpu.VMEM((K * 4,), jnp.float32),
            pltpu.VMEM((ROWS_PER_W,), jnp.int32),
            pltpu.VMEM((ELEMS_PER_W,), jnp.float32),
        ],
        compiler_params=pltpu.CompilerParams(needs_layout_passes=False),
    )
    return f(ti_pad, br_flat)


def kernel(batch_idx, topk_idx, box_regression):
    ti = topk_idx[0].astype(jnp.int32)
    ti_pad = jnp.concatenate([ti, jnp.zeros((KPAD - K,), jnp.int32)])
    br_flat = box_regression[0, :K, :].reshape(-1)
    out = _run(ti_pad, br_flat)
    return out.reshape(KPAD, 4)[:K][None]


# speedup vs baseline: 1.2210x; 1.0358x over previous
import jax
import jax.numpy as jnp
from jax import lax
from jax.experimental import pallas as pl
from jax.experimental.pallas import tpu as pltpu
from jax.experimental.pallas import tpu_sc as plsc

K = 20000
NW = 32
ROWS_PER_W = 640
TAIL_ROWS = K - (NW - 1) * ROWS_PER_W
CHUNK = 128


def _body(ti_hbm, tab_hbm, out_hbm, ti_v, idx2_v, idxg_v, rows_v, out_v, sem):
    nc = 2
    wid = lax.axis_index("s") * nc + lax.axis_index("c")
    base = wid * ROWS_PER_W
    pltpu.sync_copy(ti_hbm, ti_v)
    lanes = lax.iota(jnp.int32, 16)
    quarter = lanes >> 2
    comp = lanes & 3

    def compose(j):
        first = ti_v[pl.ds(base + j * 16, 16)]
        sec = plsc.load_gather(ti_v, [first])
        idx2_v[pl.ds(j * 16, 16)] = sec
        idxg_v[j // 8, pl.ds((j % 8) * 16, 16)] = sec >> 2

    def extract(j):
        row_local = j * 4 + quarter
        full = plsc.load_gather(idx2_v, [row_local])
        val = plsc.load_gather(rows_v, [row_local, (full & 3) * 4 + comp])
        out_v[pl.ds(j * 16, 16)] = val

    @pl.when(wid < NW - 1)
    def _full():
        for j in range(ROWS_PER_W // 16):
            compose(j)
        cps = [
            pltpu.async_copy(
                tab_hbm.at[idxg_v.at[t]],
                rows_v.at[pl.ds(t * CHUNK, CHUNK)],
                sem,
            )
            for t in range(ROWS_PER_W // CHUNK)
        ]
        for cp in cps:
            cp.wait()
        for j in range(ROWS_PER_W // 4):
            extract(j)
        pltpu.sync_copy(out_v, out_hbm.at[pl.ds(base * 4, ROWS_PER_W * 4)])

    @pl.when(wid == NW - 1)
    def _tail():
        for j in range(TAIL_ROWS // 16):
            compose(j)
        cps = [
            pltpu.async_copy(
                tab_hbm.at[idxg_v.at[0]], rows_v.at[pl.ds(0, CHUNK)], sem
            ),
            pltpu.async_copy(
                tab_hbm.at[idxg_v.at[1, pl.ds(0, TAIL_ROWS - CHUNK)]],
                rows_v.at[pl.ds(CHUNK, TAIL_ROWS - CHUNK)],
                sem,
            ),
        ]
        for cp in cps:
            cp.wait()
        for j in range(TAIL_ROWS // 4):
            extract(j)
        pltpu.sync_copy(
            out_v.at[pl.ds(0, TAIL_ROWS * 4)],
            out_hbm.at[pl.ds(base * 4, TAIL_ROWS * 4)],
        )


@jax.jit
def _run(ti, tab):
    mesh = plsc.VectorSubcoreMesh(
        core_axis_name="c", subcore_axis_name="s", num_cores=2, num_subcores=16
    )
    f = pl.kernel(
        _body,
        out_type=jax.ShapeDtypeStruct((K * 4,), jnp.float32),
        mesh=mesh,
        scratch_types=[
            pltpu.VMEM((K,), jnp.int32),
            pltpu.VMEM((ROWS_PER_W,), jnp.int32),
            pltpu.VMEM((ROWS_PER_W // CHUNK, CHUNK), jnp.int32),
            pltpu.VMEM((ROWS_PER_W, 16), jnp.float32),
            pltpu.VMEM((ROWS_PER_W * 4,), jnp.float32),
            pltpu.SemaphoreType.DMA,
        ],
        compiler_params=pltpu.CompilerParams(
            needs_layout_passes=False, use_tc_tiling_on_sc=False
        ),
    )
    return f(ti, tab)


def kernel(batch_idx, topk_idx, box_regression):
    tab = box_regression[0, :K, :].reshape(K // 4, 16)
    out = _run(topk_idx[0].astype(jnp.int32), tab)
    return out.reshape(1, K, 4)


# speedup vs baseline: 1.4554x; 1.1920x over previous
import jax
import jax.numpy as jnp
from jax import lax
from jax.experimental import pallas as pl
from jax.experimental.pallas import tpu as pltpu
from jax.experimental.pallas import tpu_sc as plsc

K = 20000
NW = 32
ROWS_PER_W = 640
TAIL_ROWS = K - (NW - 1) * ROWS_PER_W
CHUNK = 128


def _body(ti_hbm, tabg_hbm, out_hbm, ti_v, idx2_v, idxg_v, rows_v, out_v, sem):
    nc = 2
    wid = lax.axis_index("s") * nc + lax.axis_index("c")
    base = wid * ROWS_PER_W
    pltpu.sync_copy(ti_hbm.at[0], ti_v)
    lanes = lax.iota(jnp.int32, 16)
    quarter = lanes >> 2
    comp = lanes & 3

    def compose(j, carry):
        first = ti_v[pl.ds(base + j * 16, 16)]
        sec = plsc.load_gather(ti_v, [first])
        idx2_v[pl.ds(j * 16, 16)] = sec
        idxg_v[j >> 3, pl.ds((j & 7) * 16, 16)] = sec >> 2
        return carry

    def extract(j, carry):
        row_local = j * 4 + quarter
        full = plsc.load_gather(idx2_v, [row_local])
        val = plsc.load_gather(rows_v, [row_local, (full & 3) * 4 + comp])
        plsc.store_scatter(out_v, [row_local, comp], val)
        return carry

    @pl.when(wid < NW - 1)
    def _full():
        lax.fori_loop(0, ROWS_PER_W // 16, compose, 0, unroll=4)
        cps = [
            pltpu.async_copy(
                tabg_hbm.at[idxg_v.at[t]],
                rows_v.at[pl.ds(t * CHUNK, CHUNK)],
                sem,
            )
            for t in range(ROWS_PER_W // CHUNK)
        ]
        for cp in cps:
            cp.wait()
        lax.fori_loop(0, ROWS_PER_W // 4, extract, 0, unroll=4)
        pltpu.sync_copy(out_v, out_hbm.at[0, pl.ds(base, ROWS_PER_W)])

    @pl.when(wid == NW - 1)
    def _tail():
        lax.fori_loop(0, TAIL_ROWS // 16, compose, 0, unroll=4)
        cps = [
            pltpu.async_copy(
                tabg_hbm.at[idxg_v.at[0]], rows_v.at[pl.ds(0, CHUNK)], sem
            ),
            pltpu.async_copy(
                tabg_hbm.at[idxg_v.at[1, pl.ds(0, TAIL_ROWS - CHUNK)]],
                rows_v.at[pl.ds(CHUNK, TAIL_ROWS - CHUNK)],
                sem,
            ),
        ]
        for cp in cps:
            cp.wait()
        lax.fori_loop(0, TAIL_ROWS // 4, extract, 0, unroll=4)
        pltpu.sync_copy(
            out_v.at[pl.ds(0, TAIL_ROWS)],
            out_hbm.at[0, pl.ds(base, TAIL_ROWS)],
        )


@jax.jit
def _run(ti, tab):
    mesh = plsc.VectorSubcoreMesh(
        core_axis_name="c", subcore_axis_name="s", num_cores=2, num_subcores=16
    )
    f = pl.kernel(
        _body,
        out_type=jax.ShapeDtypeStruct((1, K, 4), jnp.float32),
        mesh=mesh,
        scratch_types=[
            pltpu.VMEM((K,), jnp.int32),
            pltpu.VMEM((ROWS_PER_W,), jnp.int32),
            pltpu.VMEM((ROWS_PER_W // CHUNK, CHUNK), jnp.int32),
            pltpu.VMEM((ROWS_PER_W, 16), jnp.float32),
            pltpu.VMEM((ROWS_PER_W, 4), jnp.float32),
            pltpu.SemaphoreType.DMA,
        ],
        compiler_params=pltpu.CompilerParams(
            needs_layout_passes=False, use_tc_tiling_on_sc=False
        ),
    )
    return f(ti, tab)


def kernel(batch_idx, topk_idx, box_regression):
    tabg = box_regression[0, :K, :].reshape(K // 4, 16)
    return _run(topk_idx.astype(jnp.int32), tabg)
